# baseline (speedup 1.0000x reference)
import functools

import jax
import jax.numpy as jnp
from jax import lax
from jax.experimental import pallas as pl
from jax.experimental.pallas import tpu as pltpu
from jax.experimental.pallas import tpu_sc as plsc

N = 10000
C = 128
B = 2048
RB = 2000

E = 320000
NW = 32
CHUNK = 128
NCH = 80
EP = NW * NCH * CHUNK
NP_SRC = N + 16
NP_DST = 10240
ZROWS = NP_DST // 16
QN = 2
QCH = NCH // QN
QS = 4


def _seg_body(x_hbm, si_hbm, di_hbm, z2_hbm, z1_hbm, ones_hbm,
              sums_hbm, cnts_hbm,
              si_v, di_v, ones_v, rows0_v, rows1_v, acc_s, cnt_s,
              gsem0, gsem1):
    c = lax.axis_index("c")
    s = lax.axis_index("s")
    pltpu.sync_copy(z2_hbm, acc_s.at[pl.ds(s * ZROWS, ZROWS)])
    pltpu.sync_copy(z1_hbm, cnt_s.at[pl.ds(s * ZROWS, ZROWS)])
    pltpu.sync_copy(ones_hbm, ones_v)
    plsc.subcore_barrier()

    nq = jnp.where(c == 0, QS, 2 * QN - QS)
    base = jnp.where(c == 0, s * (QS * QCH),
                     16 * QS * QCH + s * ((2 * QN - QS) * QCH))

    def quarter(q, carry):
        pltpu.sync_copy(si_hbm.at[pl.ds(base + q * QCH, QCH)], si_v)
        pltpu.sync_copy(di_hbm.at[pl.ds(base + q * QCH, QCH)], di_v)
        pltpu.async_copy(x_hbm.at[si_v.at[0]], rows0_v, gsem0)

        def body(i, carry2):
            j0 = 2 * i
            pltpu.async_copy(x_hbm.at[si_v.at[j0 + 1]], rows1_v, gsem1)
            pltpu.make_async_copy(x_hbm.at[si_v.at[0]], rows0_v, gsem0).wait()
            pltpu.sync_copy(rows0_v, acc_s.at[di_v.at[j0]], add=True)
            pltpu.sync_copy(ones_v, cnt_s.at[di_v.at[j0]], add=True)

            @pl.when(i + 1 < QCH // 2)
            def _():
                pltpu.async_copy(x_hbm.at[si_v.at[j0 + 2]], rows0_v, gsem0)

            pltpu.make_async_copy(x_hbm.at[si_v.at[0]], rows1_v, gsem1).wait()
            pltpu.sync_copy(rows1_v, acc_s.at[di_v.at[j0 + 1]], add=True)
            pltpu.sync_copy(ones_v, cnt_s.at[di_v.at[j0 + 1]], add=True)
            return carry2

        lax.fori_loop(0, QCH // 2, body, 0)
        return carry

    lax.fori_loop(0, nq, quarter, 0)
    plsc.subcore_barrier()
    pltpu.sync_copy(acc_s.at[pl.ds(s * ZROWS, ZROWS)],
                    sums_hbm.at[c].at[pl.ds(s * ZROWS, ZROWS)])
    pltpu.sync_copy(cnt_s.at[pl.ds(s * ZROWS, ZROWS)],
                    cnts_hbm.at[c].at[pl.ds(s * ZROWS, ZROWS)])


def _seg_sum_sc(xp, si2, di2, z2, z1, ones1):
    mesh = plsc.VectorSubcoreMesh(core_axis_name="c", subcore_axis_name="s")
    kfn = pl.kernel(
        _seg_body,
        out_type=[jax.ShapeDtypeStruct((2, NP_DST, C), jnp.float32),
                  jax.ShapeDtypeStruct((2, NP_DST), jnp.float32)],
        mesh=mesh,
        scratch_types=[
            pltpu.VMEM((QCH, CHUNK), jnp.int32),
            pltpu.VMEM((QCH, CHUNK), jnp.int32),
            pltpu.VMEM((CHUNK,), jnp.float32),
            pltpu.VMEM((CHUNK, C), jnp.float32),
            pltpu.VMEM((CHUNK, C), jnp.float32),
            pltpu.VMEM_SHARED((NP_DST, C), jnp.float32),
            pltpu.VMEM_SHARED((NP_DST,), jnp.float32),
            pltpu.SemaphoreType.DMA,
            pltpu.SemaphoreType.DMA,
        ],
    )
    return kfn(xp, si2, di2, z2, z1, ones1)


def _prep_edges(ei):
    src = jnp.pad(ei[0].astype(jnp.int32), (0, EP - E), constant_values=N + 8)
    dst = jnp.pad(ei[1].astype(jnp.int32), (0, EP - E), constant_values=NP_DST - 8)
    return src, dst


def _enc_body(tf_ref, W_ref, b_ref, rel_ref, wt_ref, bt_ref, o_ref):
    acc = jnp.dot(tf_ref[...], W_ref[...], preferred_element_type=jnp.float32)
    o_ref[...] = acc + b_ref[...] + bt_ref[...] + rel_ref[...] * wt_ref[...]


def _encode(tf, W, b, rel, wt, bt):
    n = tf.shape[0]
    grid = n // RB
    return pl.pallas_call(
        _enc_body,
        grid=(grid,),
        in_specs=[
            pl.BlockSpec((RB, C), lambda i: (i, 0)),
            pl.BlockSpec((C, C), lambda i: (0, 0)),
            pl.BlockSpec((1, C), lambda i: (0, 0)),
            pl.BlockSpec((RB, 1), lambda i: (i, 0)),
            pl.BlockSpec((1, C), lambda i: (0, 0)),
            pl.BlockSpec((1, C), lambda i: (0, 0)),
        ],
        out_specs=pl.BlockSpec((RB, C), lambda i: (i, 0)),
        out_shape=jax.ShapeDtypeStruct((n, C), jnp.float32),
    )(tf, W, b.reshape(1, C), rel, wt, bt.reshape(1, C))


def _sage_body(x_ref, s0_ref, s1_ref, cnt_ref, Ws_ref, Wn_ref, b_ref, o_ref):
    cnt = jnp.maximum(cnt_ref[...][:, 0:1] + cnt_ref[...][:, 1:2], 1.0)
    agg = (s0_ref[...] + s1_ref[...]) / cnt
    acc = jnp.dot(x_ref[...], Ws_ref[...], preferred_element_type=jnp.float32)
    acc = acc + jnp.dot(agg, Wn_ref[...], preferred_element_type=jnp.float32)
    o_ref[...] = jnp.maximum(acc + b_ref[...], 0.0)


def _sage(x, s0, s1, cntT, Ws, Wn, b, rb=RB):
    n = x.shape[0]
    grid = n // rb
    return pl.pallas_call(
        _sage_body,
        grid=(grid,),
        in_specs=[
            pl.BlockSpec((rb, C), lambda i: (i, 0)),
            pl.BlockSpec((rb, C), lambda i: (i, 0)),
            pl.BlockSpec((rb, C), lambda i: (i, 0)),
            pl.BlockSpec((rb, 2), lambda i: (i, 0)),
            pl.BlockSpec((C, C), lambda i: (0, 0)),
            pl.BlockSpec((C, C), lambda i: (0, 0)),
            pl.BlockSpec((1, C), lambda i: (0, 0)),
        ],
        out_specs=pl.BlockSpec((rb, C), lambda i: (i, 0)),
        out_shape=jax.ShapeDtypeStruct((n, C), jnp.float32),
    )(x, s0, s1, cntT, Ws, Wn, b.reshape(1, C))


def _head_body(x_ref, s0_ref, s1_ref, cnt_ref, Ws_ref, Wn_ref, b_ref,
               whT_ref, bh_ref, o_ref):
    cnt = jnp.maximum(cnt_ref[...][:, 0:1] + cnt_ref[...][:, 1:2], 1.0)
    agg = (s0_ref[...] + s1_ref[...]) / cnt
    acc = jnp.dot(x_ref[...], Ws_ref[...], preferred_element_type=jnp.float32)
    acc = acc + jnp.dot(agg, Wn_ref[...], preferred_element_type=jnp.float32)
    h = jnp.maximum(acc + b_ref[...], 0.0)
    o_ref[...] = jnp.sum(h * whT_ref[...], axis=1, keepdims=True) + bh_ref[0, 0]


def _sage_head(x, s0, s1, cntT, Ws, Wn, b, W_head, b_head):
    return pl.pallas_call(
        _head_body,
        grid=(1,),
        in_specs=[
            pl.BlockSpec((B, C), lambda i: (0, 0)),
            pl.BlockSpec((B, C), lambda i: (0, 0)),
            pl.BlockSpec((B, C), lambda i: (0, 0)),
            pl.BlockSpec((B, 2), lambda i: (0, 0)),
            pl.BlockSpec((C, C), lambda i: (0, 0)),
            pl.BlockSpec((C, C), lambda i: (0, 0)),
            pl.BlockSpec((1, C), lambda i: (0, 0)),
            pl.BlockSpec((1, C), lambda i: (0, 0)),
            pl.BlockSpec((1, 1), lambda i: (0, 0)),
        ],
        out_specs=pl.BlockSpec((B, 1), lambda i: (0, 0)),
        out_shape=jax.ShapeDtypeStruct((B, 1), jnp.float32),
    )(x, s0, s1, cntT, Ws, Wn, b.reshape(1, C), W_head.reshape(1, C),
      b_head.reshape(1, 1))


def kernel(tf_user, tf_item, edge_index_u2i, edge_index_i2u, seed_time,
           time_user, time_item, batch_user, batch_item,
           W_enc_user, b_enc_user, W_enc_item, b_enc_item,
           W_time_user, b_time_user, W_time_item, b_time_item,
           W_self_user_0, W_nbr_i2u_0, b_user_0,
           W_self_item_0, W_nbr_u2i_0, b_item_0,
           W_self_user_1, W_nbr_i2u_1, b_user_1,
           W_self_item_1, W_nbr_u2i_1, b_item_1,
           W_head, b_head):
    rel_u = (seed_time[batch_user] - time_user).astype(jnp.float32)[:, None] / 86400.0
    rel_i = (seed_time[batch_item] - time_item).astype(jnp.float32)[:, None] / 86400.0
    x_u = _encode(tf_user, W_enc_user, b_enc_user, rel_u, W_time_user, b_time_user)
    x_i = _encode(tf_item, W_enc_item, b_enc_item, rel_i, W_time_item, b_time_item)

    z2 = jnp.zeros((ZROWS, C), jnp.float32)
    z1 = jnp.zeros((ZROWS,), jnp.float32)
    ones1 = jnp.ones((CHUNK,), jnp.float32)
    si_i2u, di_i2u = _prep_edges(edge_index_i2u)
    si_u2i, di_u2i = _prep_edges(edge_index_u2i)
    si2_i2u = si_i2u.reshape(-1, CHUNK)
    di2_i2u = di_i2u.reshape(-1, CHUNK)
    pad_rows = ((0, NP_SRC - N), (0, 0))

    xp_i = jnp.pad(x_i, pad_rows)
    su0, cu0 = _seg_sum_sc(xp_i, si2_i2u, di2_i2u, z2, z1, ones1)
    si0, ci0 = _seg_sum_sc(jnp.pad(x_u, pad_rows),
                           si_u2i.reshape(-1, CHUNK), di_u2i.reshape(-1, CHUNK),
                           z2, z1, ones1)
    x_u1 = _sage(x_u[:B], su0[0, :B], su0[1, :B], cu0.T[:B],
                 W_self_user_0, W_nbr_i2u_0, b_user_0, rb=B)
    x_i1 = _sage(x_i, si0[0, :N], si0[1, :N], ci0.T[:N],
                 W_self_item_0, W_nbr_u2i_0, b_item_0)

    su1, cu1 = _seg_sum_sc(jnp.pad(x_i1, pad_rows), si2_i2u, di2_i2u,
                            z2, z1, ones1)
    return _sage_head(x_u1, su1[0, :B], su1[1, :B], cu1.T[:B],
                      W_self_user_1, W_nbr_i2u_1, b_user_1, W_head, b_head)


# speedup vs baseline: 1.1746x; 1.1746x over previous
import jax
import jax.numpy as jnp
from jax import lax
from jax.experimental import pallas as pl
from jax.experimental.pallas import tpu as pltpu
from jax.experimental.pallas import tpu_sc as plsc

N = 10000
C = 128
B = 2048
RB = 2000

E = 320000
NW = 32
CHUNK = 128
NCH = 80
EP = NW * NCH * CHUNK
NP_SRC = N + 16
NP_DST = 10240
ZROWS = NP_DST // 16
QN = 2
QCH = NCH // QN
QS = 3


def _seg_body(x_hbm, si_hbm, di_hbm, z2_hbm, z1_hbm, ones_hbm,
              sums_hbm, cnts_hbm,
              si_v, di_v, ones_v, rows0_v, rows1_v, acc_s, cnt_s,
              gsem0, gsem1):
    c = lax.axis_index("c")
    s = lax.axis_index("s")
    pltpu.sync_copy(z2_hbm, acc_s.at[pl.ds(s * ZROWS, ZROWS)])
    pltpu.sync_copy(z1_hbm, cnt_s.at[pl.ds(s * ZROWS, ZROWS)])
    pltpu.sync_copy(ones_hbm, ones_v)
    plsc.subcore_barrier()

    nq = jnp.where(c == 0, QS, 2 * QN - QS)
    base = jnp.where(c == 0, s * (QS * QCH),
                     16 * QS * QCH + s * ((2 * QN - QS) * QCH))

    def quarter(q, carry):
        pltpu.sync_copy(si_hbm.at[pl.ds(base + q * QCH, QCH)], si_v)
        pltpu.sync_copy(di_hbm.at[pl.ds(base + q * QCH, QCH)], di_v)
        pltpu.async_copy(x_hbm.at[si_v.at[0]], rows0_v, gsem0)

        def body(i, carry2):
            j0 = 2 * i
            pltpu.async_copy(x_hbm.at[si_v.at[j0 + 1]], rows1_v, gsem1)
            pltpu.make_async_copy(x_hbm.at[si_v.at[0]], rows0_v, gsem0).wait()
            pltpu.sync_copy(rows0_v, acc_s.at[di_v.at[j0]], add=True)
            pltpu.sync_copy(ones_v, cnt_s.at[di_v.at[j0]], add=True)

            @pl.when(i + 1 < QCH // 2)
            def _():
                pltpu.async_copy(x_hbm.at[si_v.at[j0 + 2]], rows0_v, gsem0)

            pltpu.make_async_copy(x_hbm.at[si_v.at[0]], rows1_v, gsem1).wait()
            pltpu.sync_copy(rows1_v, acc_s.at[di_v.at[j0 + 1]], add=True)
            pltpu.sync_copy(ones_v, cnt_s.at[di_v.at[j0 + 1]], add=True)
            return carry2

        lax.fori_loop(0, QCH // 2, body, 0)
        return carry

    lax.fori_loop(0, nq, quarter, 0)
    plsc.subcore_barrier()
    pltpu.sync_copy(acc_s.at[pl.ds(s * ZROWS, ZROWS)],
                    sums_hbm.at[c].at[pl.ds(s * ZROWS, ZROWS)])
    pltpu.sync_copy(cnt_s.at[pl.ds(s * ZROWS, ZROWS)],
                    cnts_hbm.at[c].at[pl.ds(s * ZROWS, ZROWS)])


def _seg_sum_sc(xp, si2, di2, z2, z1, ones1):
    mesh = plsc.VectorSubcoreMesh(core_axis_name="c", subcore_axis_name="s")
    kfn = pl.kernel(
        _seg_body,
        out_type=[jax.ShapeDtypeStruct((2, NP_DST, C), jnp.float32),
                  jax.ShapeDtypeStruct((2, NP_DST), jnp.float32)],
        mesh=mesh,
        scratch_types=[
            pltpu.VMEM((QCH, CHUNK), jnp.int32),
            pltpu.VMEM((QCH, CHUNK), jnp.int32),
            pltpu.VMEM((CHUNK,), jnp.float32),
            pltpu.VMEM((CHUNK, C), jnp.float32),
            pltpu.VMEM((CHUNK, C), jnp.float32),
            pltpu.VMEM_SHARED((NP_DST, C), jnp.float32),
            pltpu.VMEM_SHARED((NP_DST,), jnp.float32),
            pltpu.SemaphoreType.DMA,
            pltpu.SemaphoreType.DMA,
        ],
    )
    return kfn(xp, si2, di2, z2, z1, ones1)


def _prep_edges(ei):
    src = jnp.pad(ei[0].astype(jnp.int32), (0, EP - E), constant_values=N + 8)
    dst = jnp.pad(ei[1].astype(jnp.int32), (0, EP - E), constant_values=NP_DST - 8)
    return src, dst


def _enc_body(tf_ref, W_ref, b_ref, rel_ref, wt_ref, bt_ref, o_ref):
    acc = jnp.dot(tf_ref[...], W_ref[...], preferred_element_type=jnp.float32)
    o_ref[...] = acc + b_ref[...] + bt_ref[...] + rel_ref[...] * wt_ref[...]


def _encode(tf, W, b, rel, wt, bt):
    n = tf.shape[0]
    grid = n // RB
    return pl.pallas_call(
        _enc_body,
        grid=(grid,),
        in_specs=[
            pl.BlockSpec((RB, C), lambda i: (i, 0)),
            pl.BlockSpec((C, C), lambda i: (0, 0)),
            pl.BlockSpec((1, C), lambda i: (0, 0)),
            pl.BlockSpec((RB, 1), lambda i: (i, 0)),
            pl.BlockSpec((1, C), lambda i: (0, 0)),
            pl.BlockSpec((1, C), lambda i: (0, 0)),
        ],
        out_specs=pl.BlockSpec((RB, C), lambda i: (i, 0)),
        out_shape=jax.ShapeDtypeStruct((n, C), jnp.float32),
    )(tf, W, b.reshape(1, C), rel, wt, bt.reshape(1, C))


def _sage_body(x_ref, s0_ref, s1_ref, cnt_ref, Ws_ref, Wn_ref, b_ref, o_ref):
    cnt = jnp.maximum(cnt_ref[...][:, 0:1] + cnt_ref[...][:, 1:2], 1.0)
    agg = (s0_ref[...] + s1_ref[...]) / cnt
    acc = jnp.dot(x_ref[...], Ws_ref[...], preferred_element_type=jnp.float32)
    acc = acc + jnp.dot(agg, Wn_ref[...], preferred_element_type=jnp.float32)
    o_ref[...] = jnp.maximum(acc + b_ref[...], 0.0)


def _sage(x, s0, s1, cntT, Ws, Wn, b, rb=RB):
    n = x.shape[0]
    grid = n // rb
    return pl.pallas_call(
        _sage_body,
        grid=(grid,),
        in_specs=[
            pl.BlockSpec((rb, C), lambda i: (i, 0)),
            pl.BlockSpec((rb, C), lambda i: (i, 0)),
            pl.BlockSpec((rb, C), lambda i: (i, 0)),
            pl.BlockSpec((rb, 2), lambda i: (i, 0)),
            pl.BlockSpec((C, C), lambda i: (0, 0)),
            pl.BlockSpec((C, C), lambda i: (0, 0)),
            pl.BlockSpec((1, C), lambda i: (0, 0)),
        ],
        out_specs=pl.BlockSpec((rb, C), lambda i: (i, 0)),
        out_shape=jax.ShapeDtypeStruct((n, C), jnp.float32),
    )(x, s0, s1, cntT, Ws, Wn, b.reshape(1, C))


def _head_body(x_ref, s0_ref, s1_ref, cnt_ref, Ws_ref, Wn_ref, b_ref,
               whT_ref, bh_ref, o_ref):
    cnt = jnp.maximum(cnt_ref[...][:, 0:1] + cnt_ref[...][:, 1:2], 1.0)
    agg = (s0_ref[...] + s1_ref[...]) / cnt
    acc = jnp.dot(x_ref[...], Ws_ref[...], preferred_element_type=jnp.float32)
    acc = acc + jnp.dot(agg, Wn_ref[...], preferred_element_type=jnp.float32)
    h = jnp.maximum(acc + b_ref[...], 0.0)
    o_ref[...] = jnp.sum(h * whT_ref[...], axis=1, keepdims=True) + bh_ref[0, 0]


def _sage_head(x, s0, s1, cntT, Ws, Wn, b, W_head, b_head):
    return pl.pallas_call(
        _head_body,
        grid=(1,),
        in_specs=[
            pl.BlockSpec((B, C), lambda i: (0, 0)),
            pl.BlockSpec((B, C), lambda i: (0, 0)),
            pl.BlockSpec((B, C), lambda i: (0, 0)),
            pl.BlockSpec((B, 2), lambda i: (0, 0)),
            pl.BlockSpec((C, C), lambda i: (0, 0)),
            pl.BlockSpec((C, C), lambda i: (0, 0)),
            pl.BlockSpec((1, C), lambda i: (0, 0)),
            pl.BlockSpec((1, C), lambda i: (0, 0)),
            pl.BlockSpec((1, 1), lambda i: (0, 0)),
        ],
        out_specs=pl.BlockSpec((B, 1), lambda i: (0, 0)),
        out_shape=jax.ShapeDtypeStruct((B, 1), jnp.float32),
    )(x, s0, s1, cntT, Ws, Wn, b.reshape(1, C), W_head.reshape(1, C),
      b_head.reshape(1, 1))


def kernel(tf_user, tf_item, edge_index_u2i, edge_index_i2u, seed_time,
           time_user, time_item, batch_user, batch_item,
           W_enc_user, b_enc_user, W_enc_item, b_enc_item,
           W_time_user, b_time_user, W_time_item, b_time_item,
           W_self_user_0, W_nbr_i2u_0, b_user_0,
           W_self_item_0, W_nbr_u2i_0, b_item_0,
           W_self_user_1, W_nbr_i2u_1, b_user_1,
           W_self_item_1, W_nbr_u2i_1, b_item_1,
           W_head, b_head):
    rel_u = (seed_time[batch_user] - time_user).astype(jnp.float32)[:, None] / 86400.0
    rel_i = (seed_time[batch_item] - time_item).astype(jnp.float32)[:, None] / 86400.0
    x_u = _encode(tf_user, W_enc_user, b_enc_user, rel_u, W_time_user, b_time_user)
    x_i = _encode(tf_item, W_enc_item, b_enc_item, rel_i, W_time_item, b_time_item)

    z2 = jnp.zeros((ZROWS, C), jnp.float32)
    z1 = jnp.zeros((ZROWS,), jnp.float32)
    ones1 = jnp.ones((CHUNK,), jnp.float32)
    si_i2u, di_i2u = _prep_edges(edge_index_i2u)
    si_u2i, di_u2i = _prep_edges(edge_index_u2i)
    si2_i2u = si_i2u.reshape(-1, CHUNK)
    di2_i2u = di_i2u.reshape(-1, CHUNK)
    pad_rows = ((0, NP_SRC - N), (0, 0))

    xp_i = jnp.pad(x_i, pad_rows)
    su0, cu0 = _seg_sum_sc(xp_i, si2_i2u, di2_i2u, z2, z1, ones1)
    si0, ci0 = _seg_sum_sc(jnp.pad(x_u, pad_rows),
                           si_u2i.reshape(-1, CHUNK), di_u2i.reshape(-1, CHUNK),
                           z2, z1, ones1)
    x_u1 = _sage(x_u[:B], su0[0, :B], su0[1, :B], cu0.T[:B],
                 W_self_user_0, W_nbr_i2u_0, b_user_0, rb=B)
    x_i1 = _sage(x_i, si0[0, :N], si0[1, :N], ci0.T[:N],
                 W_self_item_0, W_nbr_u2i_0, b_item_0)

    su1, cu1 = _seg_sum_sc(jnp.pad(x_i1, pad_rows), si2_i2u, di2_i2u,
                            z2, z1, ones1)
    return _sage_head(x_u1, su1[0, :B], su1[1, :B], cu1.T[:B],
                      W_self_user_1, W_nbr_i2u_1, b_user_1, W_head, b_head)
